# Initial kernel scaffold; baseline (speedup 1.0000x reference)
import jax
import jax.numpy as jnp
from jax.experimental import pallas as pl


def kernel(x, edge_attr, atom_table, bond_table):
    raise NotImplementedError("write your pallas kernel here")


# speedup vs baseline: 1.9927x; 1.9927x over previous
import functools

import jax
import jax.numpy as jnp
from jax import lax
from jax.experimental import pallas as pl
from jax.experimental.pallas import tpu as pltpu
from jax.experimental.pallas import tpu_sc as plsc

NC, NS = 2, 16
NW = NC * NS
CHUNK = 120

N_NODES, NUM_ATOM_FEAT, ATOM_DIM = 50000, 9, 128
N_EDGES, NUM_BOND_FEAT, BOND_DIM = 800000, 3, 16
A_ROWS = N_NODES * NUM_ATOM_FEAT
B_ROWS = N_EDGES * NUM_BOND_FEAT
A_CHUNKS = A_ROWS // CHUNK
B_CHUNKS = B_ROWS // CHUNK
A_ITERS = (A_CHUNKS + NW - 1) // NW
B_ITERS = B_CHUNKS // NW


def _body(xa_hbm, xb_hbm, at_hbm, bt_hbm, oa_hbm, ob_hbm,
          idx_a, rows_a, idx_b, rows_b, sem):
    wid = lax.axis_index("s") * NC + lax.axis_index("c")

    def do_chunk(table, idx_hbm, out_hbm, idx_v, rows_v, c):
        base = c * CHUNK
        pltpu.sync_copy(idx_hbm.at[pl.ds(base, CHUNK)], idx_v)
        pltpu.async_copy(table.at[idx_v], rows_v, sem).wait()
        pltpu.sync_copy(rows_v, out_hbm.at[pl.ds(base, CHUNK)])

    def a_step(i, carry):
        c = wid + i * NW

        @pl.when(c < A_CHUNKS)
        def _():
            do_chunk(at_hbm, xa_hbm, oa_hbm, idx_a, rows_a, c)
        return carry

    lax.fori_loop(0, A_ITERS, a_step, 0)

    def b_step(i, carry):
        do_chunk(bt_hbm, xb_hbm, ob_hbm, idx_b, rows_b, wid + i * NW)
        return carry

    lax.fori_loop(0, B_ITERS, b_step, 0)


@jax.jit
def kernel(x, edge_attr, atom_table, bond_table):
    xa = x.reshape(A_ROWS)
    xb = edge_attr.reshape(B_ROWS)
    mesh = plsc.VectorSubcoreMesh(core_axis_name="c", subcore_axis_name="s")
    run = pl.kernel(
        _body,
        out_type=(
            jax.ShapeDtypeStruct((A_ROWS, ATOM_DIM), jnp.float32),
            jax.ShapeDtypeStruct((B_ROWS, BOND_DIM), jnp.float32),
        ),
        mesh=mesh,
        scratch_types=[
            pltpu.VMEM((CHUNK,), jnp.int32),
            pltpu.VMEM((CHUNK, ATOM_DIM), jnp.float32),
            pltpu.VMEM((CHUNK,), jnp.int32),
            pltpu.VMEM((CHUNK, BOND_DIM), jnp.float32),
            pltpu.SemaphoreType.DMA,
        ],
        compiler_params=pltpu.CompilerParams(use_tc_tiling_on_sc=False),
    )
    oa, ob = run(xa, xb, atom_table, bond_table)
    return (oa.reshape(N_NODES, NUM_ATOM_FEAT, ATOM_DIM),
            ob.reshape(N_EDGES, NUM_BOND_FEAT, BOND_DIM))
